# Initial kernel scaffold; baseline (speedup 1.0000x reference)
import jax
import jax.numpy as jnp
from jax.experimental import pallas as pl


def kernel(x, eye):
    raise NotImplementedError("write your pallas kernel here")


# speedup vs baseline: 1.1384x; 1.1384x over previous
import functools

import jax
import jax.numpy as jnp
from jax import lax
from jax.experimental import pallas as pl
from jax.experimental.pallas import tpu as pltpu
from jax.experimental.pallas import tpu_sc as plsc

NUM_CLASSES = 1000
ROWS = 1024 * 26
NC, NS, L = 2, 16, 16
NW = NC * NS
ROWS_PER_W = ROWS // NW
CHUNK = 64
N_CHUNKS = ROWS_PER_W // CHUNK
BUF = CHUNK * NUM_CLASSES


def _body(x_hbm, out_hbm, idx_v, buf_v, sem):
    wid = lax.axis_index("c") * NS + lax.axis_index("s")
    base = wid * ROWS_PER_W

    pltpu.sync_copy(x_hbm.at[pl.ds(base, ROWS_PER_W)], idx_v)

    zeros = jnp.zeros((L,), jnp.float32)
    ones = jnp.ones((L,), jnp.float32)
    lane = lax.iota(jnp.int32, L)

    def zero_all(i, _):
        buf_v[pl.ds(i * L, L)] = zeros
        return _
    lax.fori_loop(0, BUF // L, zero_all, 0)

    def chunk_body(ci, _):
        row0 = ci * CHUNK
        for j in range(CHUNK // L):
            cols = idx_v[pl.ds(row0 + j * L, L)]
            flat = (lane + j * L) * NUM_CLASSES + cols
            plsc.store_scatter(buf_v, [flat], ones)
        pltpu.sync_copy(buf_v, out_hbm.at[pl.ds((base + row0) * NUM_CLASSES, BUF)])
        for j in range(CHUNK // L):
            cols = idx_v[pl.ds(row0 + j * L, L)]
            flat = (lane + j * L) * NUM_CLASSES + cols
            plsc.store_scatter(buf_v, [flat], zeros)
        return _
    lax.fori_loop(0, N_CHUNKS, chunk_body, 0)


@functools.partial(jax.jit, static_argnames=())
def kernel(x, eye):
    del eye
    flat_idx = x.reshape(ROWS)
    mesh = plsc.VectorSubcoreMesh(core_axis_name="c", subcore_axis_name="s")
    k = pl.kernel(
        _body,
        out_type=jax.ShapeDtypeStruct((ROWS * NUM_CLASSES,), jnp.float32),
        mesh=mesh,
        scratch_types=[
            pltpu.VMEM((ROWS_PER_W,), jnp.int32),
            pltpu.VMEM((BUF,), jnp.float32),
            pltpu.SemaphoreType.DMA,
        ],
        compiler_params=pltpu.CompilerParams(needs_layout_passes=False),
    )
    out = k(flat_idx)
    return out.reshape(x.shape[0], x.shape[1], NUM_CLASSES)


# speedup vs baseline: 2.0456x; 1.7969x over previous
import functools

import jax
import jax.numpy as jnp
from jax import lax
from jax.experimental import pallas as pl
from jax.experimental.pallas import tpu as pltpu
from jax.experimental.pallas import tpu_sc as plsc

B, S = 1024, 26
NUM_CLASSES = 1000
NC, NS, L = 2, 16, 16
NW = NC * NS
B_PER_W = B // NW
CHUNK_B = 2
CHUNK_R = CHUNK_B * S
N_CHUNKS = B_PER_W // CHUNK_B
IDX_PAD = 64
IDX_PER_W = N_CHUNKS * IDX_PAD


def _body(x_hbm, out_hbm, idx_v, buf_v, sem):
    wid = lax.axis_index("c") * NS + lax.axis_index("s")
    batch0 = wid * B_PER_W

    pltpu.sync_copy(x_hbm.at[pl.ds(wid * IDX_PER_W, IDX_PER_W)], idx_v)

    zeros = jnp.zeros((L,), jnp.float32)
    ones = jnp.ones((L,), jnp.float32)
    lane = lax.iota(jnp.int32, L)

    def zero_row(r, _):
        b = r // S
        rr = r % S
        def zero_slice(i, _):
            buf_v[b, rr, pl.ds(i * L, L)] = zeros
            return _
        lax.fori_loop(0, NUM_CLASSES // L, zero_slice, 0)
        buf_v[b, rr, pl.ds(NUM_CLASSES - L, L)] = zeros
        return _
    lax.fori_loop(0, CHUNK_R, zero_row, 0)

    def chunk_body(ci, _):
        def scatter(vals):
            for j in range(CHUNK_R // L + 1):
                r = lane + j * L
                mask = r < CHUNK_R
                cols = idx_v[pl.ds(ci * IDX_PAD + j * L, L)]
                plsc.store_scatter(buf_v, [r // S, r % S, cols], vals, mask=mask)
        scatter(ones)
        pltpu.sync_copy(buf_v, out_hbm.at[pl.ds(batch0 + ci * CHUNK_B, CHUNK_B)])
        scatter(zeros)
        return _
    lax.fori_loop(0, N_CHUNKS, chunk_body, 0)


@functools.partial(jax.jit, static_argnames=())
def kernel(x, eye):
    del eye
    xr = x.reshape(NW, N_CHUNKS, CHUNK_R)
    xp = jnp.pad(xr, ((0, 0), (0, 0), (0, IDX_PAD - CHUNK_R))).reshape(-1)
    mesh = plsc.VectorSubcoreMesh(core_axis_name="c", subcore_axis_name="s")
    k = pl.kernel(
        _body,
        out_type=jax.ShapeDtypeStruct((B, S, NUM_CLASSES), jnp.float32),
        mesh=mesh,
        scratch_types=[
            pltpu.VMEM((IDX_PER_W,), jnp.int32),
            pltpu.VMEM((CHUNK_B, S, NUM_CLASSES), jnp.float32),
            pltpu.SemaphoreType.DMA,
        ],
        compiler_params=pltpu.CompilerParams(needs_layout_passes=False),
    )
    return k(xp)
